# baseline (speedup 1.0000x reference)
import jax
import jax.numpy as jnp
from jax.experimental import pallas as pl
from jax.experimental.pallas import tpu as pltpu

_N = 4096
_BLK = 256
_GRID = _N // _BLK
_BLK2 = 512
_GRID2 = _N // _BLK2


def _prep_body(adj_ref, diag_ref, adj8_ref, dsl_ref):
    a = adj_ref[...]
    sub = diag_ref[...]
    eye = (jax.lax.broadcasted_iota(jnp.int32, (_BLK, _BLK), 0)
           == jax.lax.broadcasted_iota(jnp.int32, (_BLK, _BLK), 1))
    diag = jnp.sum(jnp.where(eye, sub, 0.0), axis=1)
    sl = jnp.where(diag == 0.0, 1.0, 0.0)
    deg = jnp.sum(a, axis=1) + sl
    dinv = jax.lax.rsqrt(deg)
    dsl_ref[...] = jnp.concatenate(
        [jnp.broadcast_to(dinv[:, None], (_BLK, 128)),
         jnp.broadcast_to(sl[:, None], (_BLK, 128))], axis=1)
    adj8_ref[...] = a.astype(jnp.bfloat16)


def _mega_body(adj8_ref, gn_ref, feat_ref, feata_ref, w1_ref, w2_ref,
               dsl_ref, w0_ref, b_ref,
               zcat_ref, h_ref, retcat_ref,
               xs_s, ys_s, embcat_s, vs_s):
    p = pl.program_id(0)
    i = pl.program_id(1)
    f32 = jnp.float32
    bf16 = jnp.bfloat16

    @pl.when((p == 0) & (i == 0))
    def _xs():
        xw = jnp.dot(feat_ref[...], w1_ref[...], preferred_element_type=f32)
        xwa = jnp.dot(feata_ref[...], w1_ref[...], preferred_element_type=f32)
        dinvf = dsl_ref[:, :128]
        xs_s[...] = (jnp.concatenate([xw, xwa], axis=1) * dinvf).astype(bf16)

    @pl.when(p == 0)
    def _prop1():
        a8 = adj8_ref[...]
        xsb = xs_s[...]
        xsi = xs_s[pl.ds(i * _BLK2, _BLK2), :].astype(f32)
        dsl = dsl_ref[pl.ds(i * _BLK2, _BLK2), :]
        dinvb = dsl[:, :128]
        slb = dsl[:, 128:]
        acc = jnp.dot(a8, xsb, preferred_element_type=f32)
        zc = (acc + slb * xsi) * dinvb
        z = zc[:, :64]
        emb = jnp.maximum(z, 0.0)
        emba = jnp.maximum(zc[:, 64:], 0.0)
        zcat_ref[...] = jnp.concatenate([z, emb, emba], axis=1)
        embcat_s[i] = jnp.concatenate(
            [emb, emba, jnp.full((_BLK2, 1), 1.0, f32),
             jnp.zeros((_BLK2, 127), f32)], axis=1).astype(bf16)
        ys = jnp.dot(z, w2_ref[...], preferred_element_type=f32) \
            * dinvb[:, :128]
        ys_s[i] = ys.astype(bf16)

    @pl.when(p == 1)
    def _readout():
        g = gn_ref[...]
        vs_s[i] = jnp.dot(g.astype(bf16), embcat_s[...].reshape(_N, 256),
                          preferred_element_type=f32)

    @pl.when(p == 2)
    def _prop2_fin():
        a8 = adj8_ref[...]
        ysb = ys_s[...].reshape(_N, 128)
        ysi = ys_s[i].astype(f32)
        dsl = dsl_ref[pl.ds(i * _BLK2, _BLK2), :]
        acc = jnp.dot(a8, ysb, preferred_element_type=f32)
        h_ref[...] = (acc + dsl[:, 128:] * ysi) * dsl[:, :128]

        vs = vs_s[i]
        gc = vs[:, :128] / vs[:, 128:129]
        gp = gc[:, :64]
        gpa = gc[:, 64:]

        def l2sig(x):
            nrm = jnp.sqrt(jnp.sum(x * x, axis=1, keepdims=True))
            return jax.nn.sigmoid(x / jnp.maximum(nrm, 1e-12))

        gp = l2sig(gp)
        gpa = l2sig(gpa)
        ec = embcat_s[i][:, :128].astype(f32)
        w0 = w0_ref[0]
        hw = jnp.dot(ec[:, :64], w0, preferred_element_type=f32)
        hwa = jnp.dot(ec[:, 64:], w0, preferred_element_type=f32)
        b = b_ref[0, 0]
        r0 = jnp.sum(hw * gp, axis=1, keepdims=True) + b
        r1 = jnp.sum(hwa * gp, axis=1, keepdims=True) + b
        ra0 = jnp.sum(hwa * gpa, axis=1, keepdims=True) + b
        ra1 = jnp.sum(hw * gpa, axis=1, keepdims=True) + b
        retcat_ref[...] = jnp.concatenate([r0, r1, ra0, ra1], axis=1)


def _const(shape):
    nd = len(shape)
    return pl.BlockSpec(shape, lambda *_, _nd=nd: (0,) * _nd)


def kernel(feat, feat_a, adj, graph_neigh, W1, W2, disc_W, disc_b):
    f32 = jnp.float32
    bf16 = jnp.bfloat16

    adj8, dsl = pl.pallas_call(
        _prep_body,
        grid=(_GRID,),
        in_specs=[
            pl.BlockSpec((_BLK, _N), lambda i: (i, 0)),
            pl.BlockSpec((_BLK, _BLK), lambda i: (i, i)),
        ],
        out_specs=[
            pl.BlockSpec((_BLK, _N), lambda i: (i, 0)),
            pl.BlockSpec((_BLK, 256), lambda i: (i, 0)),
        ],
        out_shape=[
            jax.ShapeDtypeStruct((_N, _N), jnp.bfloat16),
            jax.ShapeDtypeStruct((_N, 256), f32),
        ],
        compiler_params=pltpu.CompilerParams(
            vmem_limit_bytes=100 * 1024 * 1024,
        ),
    )(adj, adj)

    def _adj8_map(p, i):
        return (jnp.where(p == 1, _GRID2 - 1, i), 0)

    def _gn_map(p, i):
        return (jnp.where(p == 0, 0, jnp.where(p == 1, i, _GRID2 - 1)), 0)

    def _p0_out(p, i):
        return (jnp.where(p == 0, i, _GRID2 - 1), 0)

    def _p2_out(p, i):
        return (jnp.where(p == 2, i, 0), 0)

    zcat, h, retcat = pl.pallas_call(
        _mega_body,
        grid=(3, _GRID2),
        in_specs=[
            pl.BlockSpec((_BLK2, _N), _adj8_map),
            pl.BlockSpec((_BLK2, _N), _gn_map),
            _const((_N, 128)),
            _const((_N, 128)),
            _const((128, 64)),
            _const((64, 128)),
            _const((_N, 256)),
            _const((1, 64, 64)),
            _const((1, 1)),
        ],
        out_specs=[
            pl.BlockSpec((_BLK2, 192), _p0_out),
            pl.BlockSpec((_BLK2, 128), _p2_out),
            pl.BlockSpec((_BLK2, 4), _p2_out),
        ],
        out_shape=[
            jax.ShapeDtypeStruct((_N, 192), f32),
            jax.ShapeDtypeStruct((_N, 128), f32),
            jax.ShapeDtypeStruct((_N, 4), f32),
        ],
        scratch_shapes=[
            pltpu.VMEM((_N, 128), bf16),
            pltpu.VMEM((_GRID2, _BLK2, 128), bf16),
            pltpu.VMEM((_GRID2, _BLK2, 256), bf16),
            pltpu.VMEM((_GRID2, _BLK2, 256), f32),
        ],
        compiler_params=pltpu.CompilerParams(
            vmem_limit_bytes=100 * 1024 * 1024,
        ),
    )(adj8, graph_neigh, feat, feat_a, W1, W2, dsl, disc_W,
      disc_b.reshape(1, 1))

    z = zcat[:, :64]
    emb = zcat[:, 64:128]
    emb_a = zcat[:, 128:]
    ret = retcat[:, :2]
    ret_a = retcat[:, 2:]
    return (z, h, ret, ret_a, emb, emb_a)


# speedup vs baseline: 1.0977x; 1.0977x over previous
import jax
import jax.numpy as jnp
from jax.experimental import pallas as pl
from jax.experimental.pallas import tpu as pltpu

_N = 4096
_BLK = 512
_GRID = _N // _BLK
_BLK2 = 512
_GRID2 = _N // _BLK2


def _prep_body(adj_ref, diag_ref, adj8_ref, dsl_ref):
    a = adj_ref[...]
    sub = diag_ref[...]
    eye = (jax.lax.broadcasted_iota(jnp.int32, (_BLK, _BLK), 0)
           == jax.lax.broadcasted_iota(jnp.int32, (_BLK, _BLK), 1))
    diag = jnp.sum(jnp.where(eye, sub, 0.0), axis=1)
    sl = jnp.where(diag == 0.0, 1.0, 0.0)
    deg = jnp.sum(a, axis=1) + sl
    dinv = jax.lax.rsqrt(deg)
    dsl_ref[...] = jnp.concatenate(
        [jnp.broadcast_to(dinv[:, None], (_BLK, 128)),
         jnp.broadcast_to(sl[:, None], (_BLK, 128))], axis=1)
    adj8_ref[...] = a.astype(jnp.int8)


def _mega_body(adj8_ref, gn_ref, feat_ref, feata_ref, w1_ref, w2_ref,
               dsl_ref, w0_ref, b_ref,
               zcat_ref, h_ref, retcat_ref,
               xs_s, ys_s, embcat_s, vs_s):
    p = pl.program_id(0)
    i = pl.program_id(1)
    f32 = jnp.float32
    bf16 = jnp.bfloat16

    @pl.when((p == 0) & (i == 0))
    def _xs():
        xw = jnp.dot(feat_ref[...], w1_ref[...], preferred_element_type=f32)
        xwa = jnp.dot(feata_ref[...], w1_ref[...], preferred_element_type=f32)
        dinvf = dsl_ref[:, :128]
        xs_s[...] = (jnp.concatenate([xw, xwa], axis=1) * dinvf).astype(bf16)

    @pl.when(p == 0)
    def _prop1():
        a8 = adj8_ref[...]
        xsb = xs_s[...]
        xsi = xs_s[pl.ds(i * _BLK2, _BLK2), :].astype(f32)
        dsl = dsl_ref[pl.ds(i * _BLK2, _BLK2), :]
        dinvb = dsl[:, :128]
        slb = dsl[:, 128:]
        acc = jnp.dot(a8, xsb, preferred_element_type=f32)
        zc = (acc + slb * xsi) * dinvb
        z = zc[:, :64]
        emb = jnp.maximum(z, 0.0)
        emba = jnp.maximum(zc[:, 64:], 0.0)
        zcat_ref[...] = jnp.concatenate([z, emb, emba], axis=1)
        embcat_s[i] = jnp.concatenate(
            [emb, emba, jnp.full((_BLK2, 1), 1.0, f32),
             jnp.zeros((_BLK2, 127), f32)], axis=1).astype(bf16)
        ys = jnp.dot(z, w2_ref[...], preferred_element_type=f32) \
            * dinvb[:, :128]
        ys_s[i] = ys.astype(bf16)

    @pl.when(p == 1)
    def _readout():
        g = gn_ref[...]
        vs_s[i] = jnp.dot(g.astype(bf16), embcat_s[...].reshape(_N, 256),
                          preferred_element_type=f32)

    @pl.when(p == 2)
    def _prop2_fin():
        a8 = adj8_ref[...]
        ysb = ys_s[...].reshape(_N, 128)
        ysi = ys_s[i].astype(f32)
        dsl = dsl_ref[pl.ds(i * _BLK2, _BLK2), :]
        acc = jnp.dot(a8, ysb, preferred_element_type=f32)
        h_ref[...] = (acc + dsl[:, 128:] * ysi) * dsl[:, :128]

        vs = vs_s[i]
        gc = vs[:, :128] / vs[:, 128:129]
        gp = gc[:, :64]
        gpa = gc[:, 64:]

        def l2sig(x):
            nrm = jnp.sqrt(jnp.sum(x * x, axis=1, keepdims=True))
            return jax.nn.sigmoid(x / jnp.maximum(nrm, 1e-12))

        gp = l2sig(gp)
        gpa = l2sig(gpa)
        ec = embcat_s[i][:, :128].astype(f32)
        w0 = w0_ref[0]
        hw = jnp.dot(ec[:, :64], w0, preferred_element_type=f32)
        hwa = jnp.dot(ec[:, 64:], w0, preferred_element_type=f32)
        b = b_ref[0, 0]
        r0 = jnp.sum(hw * gp, axis=1, keepdims=True) + b
        r1 = jnp.sum(hwa * gp, axis=1, keepdims=True) + b
        ra0 = jnp.sum(hwa * gpa, axis=1, keepdims=True) + b
        ra1 = jnp.sum(hw * gpa, axis=1, keepdims=True) + b
        retcat_ref[...] = jnp.concatenate([r0, r1, ra0, ra1], axis=1)


def _const(shape):
    nd = len(shape)
    return pl.BlockSpec(shape, lambda *_, _nd=nd: (0,) * _nd)


def kernel(feat, feat_a, adj, graph_neigh, W1, W2, disc_W, disc_b):
    f32 = jnp.float32
    bf16 = jnp.bfloat16

    adj8, dsl = pl.pallas_call(
        _prep_body,
        grid=(_GRID,),
        in_specs=[
            pl.BlockSpec((_BLK, _N), lambda i: (i, 0)),
            pl.BlockSpec((_BLK, _BLK), lambda i: (i, i)),
        ],
        out_specs=[
            pl.BlockSpec((_BLK, _N), lambda i: (i, 0)),
            pl.BlockSpec((_BLK, 256), lambda i: (i, 0)),
        ],
        out_shape=[
            jax.ShapeDtypeStruct((_N, _N), jnp.int8),
            jax.ShapeDtypeStruct((_N, 256), f32),
        ],
        compiler_params=pltpu.CompilerParams(
            vmem_limit_bytes=100 * 1024 * 1024,
        ),
    )(adj, adj)

    def _adj8_map(p, i):
        return (jnp.where(p == 1, _GRID2 - 1, i), 0)

    def _gn_map(p, i):
        return (jnp.where(p == 0, 0, jnp.where(p == 1, i, _GRID2 - 1)), 0)

    def _p0_out(p, i):
        return (jnp.where(p == 0, i, _GRID2 - 1), 0)

    def _p2_out(p, i):
        return (jnp.where(p == 2, i, 0), 0)

    zcat, h, retcat = pl.pallas_call(
        _mega_body,
        grid=(3, _GRID2),
        in_specs=[
            pl.BlockSpec((_BLK2, _N), _adj8_map),
            pl.BlockSpec((_BLK2, _N), _gn_map),
            _const((_N, 128)),
            _const((_N, 128)),
            _const((128, 64)),
            _const((64, 128)),
            _const((_N, 256)),
            _const((1, 64, 64)),
            _const((1, 1)),
        ],
        out_specs=[
            pl.BlockSpec((_BLK2, 192), _p0_out),
            pl.BlockSpec((_BLK2, 128), _p2_out),
            pl.BlockSpec((_BLK2, 4), _p2_out),
        ],
        out_shape=[
            jax.ShapeDtypeStruct((_N, 192), f32),
            jax.ShapeDtypeStruct((_N, 128), f32),
            jax.ShapeDtypeStruct((_N, 4), f32),
        ],
        scratch_shapes=[
            pltpu.VMEM((_N, 128), bf16),
            pltpu.VMEM((_GRID2, _BLK2, 128), bf16),
            pltpu.VMEM((_GRID2, _BLK2, 256), bf16),
            pltpu.VMEM((_GRID2, _BLK2, 256), f32),
        ],
        compiler_params=pltpu.CompilerParams(
            vmem_limit_bytes=100 * 1024 * 1024,
        ),
    )(adj8, graph_neigh, feat, feat_a, W1, W2, dsl, disc_W,
      disc_b.reshape(1, 1))

    z = zcat[:, :64]
    emb = zcat[:, 64:128]
    emb_a = zcat[:, 128:]
    ret = retcat[:, :2]
    ret_a = retcat[:, 2:]
    return (z, h, ret, ret_a, emb, emb_a)
